# baseline (speedup 1.0000x reference)
import functools

import jax
import jax.numpy as jnp
from jax import lax
from jax.experimental import pallas as pl
from jax.experimental.pallas import tpu as pltpu
from jax.experimental.pallas import tpu_sc as plsc

B, N, E = 4, 8192, 768
NKEEP = 4915
KPAD = 5120
NH = 7
NHEADS = 4

RB = 512

_NC, _NS = 2, 16
NW = _NC * _NS
TOK_W = (B * N) // NW
ROWS_W = (B * KPAD) // NW
CH = 64
NCHUNK = ROWS_W // CH


def _rank_kernel(s_ref, st_ref, out_ref):
    i = pl.program_id(1)
    j = pl.program_id(2)
    mi = lax.bitcast_convert_type(st_ref[0], jnp.int32)
    mj = lax.bitcast_convert_type(s_ref[0], jnp.int32)

    @pl.when(j == 0)
    def _():
        out_ref[...] = jnp.zeros_like(out_ref)

    @pl.when(j < i)
    def _():
        cnt = jnp.sum(jnp.where(mj >= mi, 1, 0), axis=1)
        out_ref[...] += cnt[None, None, :]

    @pl.when(j > i)
    def _():
        cnt = jnp.sum(jnp.where(mj > mi, 1, 0), axis=1)
        out_ref[...] += cnt[None, None, :]

    @pl.when(j == i)
    def _():
        iidx = lax.broadcasted_iota(jnp.int32, (RB, RB), 0)
        jidx = lax.broadcasted_iota(jnp.int32, (RB, RB), 1)
        before = (mj > mi) | ((mj == mi) & (jidx < iidx))
        cnt = jnp.sum(jnp.where(before, 1, 0), axis=1)
        out_ref[...] += cnt[None, None, :]


def _sc_scatter_body(rank, gidx, rv, pos_v, val_v, sem):
    wid = lax.axis_index("s") * _NC + lax.axis_index("c")
    bi = wid // (NW // B)
    base = wid * TOK_W

    for r in range(TOK_W // 128):
        pltpu.sync_copy(rank.at[pl.ds(base + r * 128, 128)], rv.at[r])
    for r in range(TOK_W // 128):
        for k in range(8):
            rvv = rv[r, pl.ds(k * 16, 16)]
            pos_v[r, pl.ds(k * 16, 16)] = rvv + bi * N
            val_v[r, pl.ds(k * 16, 16)] = (
                lax.iota(jnp.int32, 16) + (base + r * 128 + k * 16))
    copies = [
        pltpu.async_copy(val_v.at[r], gidx.at[pos_v.at[r]], sem)
        for r in range(TOK_W // 128)
    ]
    for c in copies:
        c.wait()


@functools.partial(
    pl.kernel,
    mesh=plsc.VectorSubcoreMesh(core_axis_name="c", subcore_axis_name="s"),
    out_type=jax.ShapeDtypeStruct((B * N,), jnp.int32),
    scratch_types=[
        pltpu.VMEM((TOK_W // 128, 128), jnp.int32),
        pltpu.VMEM((TOK_W // 128, 128), jnp.int32),
        pltpu.VMEM((TOK_W // 128, 128), jnp.int32),
        pltpu.SemaphoreType.DMA,
    ],
)
def _sc_scatter(rank, gidx, *scratch):
    _sc_scatter_body(rank, gidx, *scratch)


def _sc_gather_body(xtile, gidx, ctab, xout, cout,
                    idx_v, jx_v, rows_v, ci0_v, ci1_v, o0_v, o1_v,
                    sem0, sem1):
    wid = lax.axis_index("s") * _NC + lax.axis_index("c")
    bi = wid // (NW // B)
    out_base = wid * ROWS_W
    q_base = (wid % (NW // B)) * ROWS_W
    sems = (sem0, sem1)

    for c in range(NCHUNK):
        pltpu.sync_copy(gidx.at[pl.ds(bi * N + q_base + c * CH, CH)],
                        idx_v.at[c])

    for c in range(NCHUNK):
        for k in range(CH // 16):
            iv = idx_v[c, pl.ds(k * 16, 16)]
            ci0_v[c, pl.ds(k * 16, 16)] = iv + bi * N
            ci1_v[c, pl.ds(k * 16, 16)] = iv + (bi + 1) * N
            bse = iv + 40 * (iv >> 3)
            for cc in range(6):
                jx_v[c, cc, pl.ds(k * 16, 16)] = bse + 8 * cc

    def fire(c):
        sem = sems[c % 2]
        ds = [
            pltpu.async_copy(xtile.at[jx_v.at[c, cc]],
                             rows_v.at[c % 2, cc], sem)
            for cc in range(6)
        ]
        ds.append(pltpu.async_copy(ctab.at[ci0_v.at[c]], o0_v.at[c], sem))
        ds.append(pltpu.async_copy(ctab.at[ci1_v.at[c]], o1_v.at[c], sem))
        return ds

    pend = fire(0)
    for c in range(NCHUNK):
        nxt = fire(c + 1) if c + 1 < NCHUNK else None
        for d in pend:
            d.wait()
        for cc in range(6):
            pltpu.sync_copy(
                rows_v.at[c % 2, cc],
                xout.at[pl.ds(out_base + c * CH, CH),
                        pl.ds(cc * 128, 128)])
        pltpu.sync_copy(
            o0_v.at[c],
            cout.at[pl.ds((bi * 2) * KPAD + q_base + c * CH, CH)])
        pltpu.sync_copy(
            o1_v.at[c],
            cout.at[pl.ds((bi * 2 + 1) * KPAD + q_base + c * CH, CH)])
        pend = nxt


@functools.partial(
    pl.kernel,
    mesh=plsc.VectorSubcoreMesh(core_axis_name="c", subcore_axis_name="s"),
    out_type=[
        jax.ShapeDtypeStruct((B * KPAD, E), jnp.float32),
        jax.ShapeDtypeStruct((B * 2 * KPAD,), jnp.float32),
    ],
    scratch_types=[
        pltpu.VMEM((NCHUNK, CH), jnp.int32),
        pltpu.VMEM((NCHUNK, 6, CH), jnp.int32),
        pltpu.VMEM((2, 6, CH, 128), jnp.float32),
        pltpu.VMEM((NCHUNK, CH), jnp.int32),
        pltpu.VMEM((NCHUNK, CH), jnp.int32),
        pltpu.VMEM((NCHUNK, CH), jnp.float32),
        pltpu.VMEM((NCHUNK, CH), jnp.float32),
        pltpu.SemaphoreType.DMA,
        pltpu.SemaphoreType.DMA,
    ],
)
def _sc_gather(xtile, gidx, ctab, xout, cout, *scratch):
    _sc_gather_body(xtile, gidx, ctab, xout, cout, *scratch)


def kernel(x, att_nh, coords, W1, b1, W2, b2, W3, b3, W4, b4, W5, b5):
    b, n, e = x.shape
    bt, n_heads, nh, _ = att_nh.shape

    a = att_nh.reshape(b, n, nh, nh, n_heads)
    h = a @ W1.T + b1
    h = h @ W2.T + b2
    att_vote = h.reshape(b, n, nh * nh)
    v = att_vote @ W3.T + b3
    v = v @ W4.T + b4
    v = v @ W5.T + b5
    v = jnp.squeeze(v)
    s = jax.nn.softmax(v, axis=1)

    st = s[:, :, None]
    s3 = s[:, None, :]

    rank = pl.pallas_call(
        _rank_kernel,
        grid=(B, N // RB, N // RB),
        in_specs=[
            pl.BlockSpec((1, 1, RB), lambda bi, i, j: (bi, 0, j)),
            pl.BlockSpec((1, RB, 1), lambda bi, i, j: (bi, i, 0)),
        ],
        out_specs=pl.BlockSpec((1, 1, RB), lambda bi, i, j: (bi, 0, i)),
        out_shape=jax.ShapeDtypeStruct((B, 1, N), jnp.int32),
    )(s3, st)

    gidx = _sc_scatter(rank.reshape(B * N))

    xtile = jnp.transpose(
        x.reshape(B * N // 8, 8, E // 128, 128), (0, 2, 1, 3)
    ).reshape(B * N // 8 * (E // 128) * 8, 128)
    ctab = coords[..., 0].reshape(B * 2 * N)
    xout, cout = _sc_gather(xtile, gidx, ctab)

    x_out = xout.reshape(B, KPAD, E)[:, :NKEEP]
    coords_out = cout.reshape(B, 2, KPAD)[:, :, :NKEEP, None]
    return (x_out, coords_out)


# speedup vs baseline: 2.1542x; 2.1542x over previous
import functools

import jax
import jax.numpy as jnp
from jax import lax
from jax.experimental import pallas as pl
from jax.experimental.pallas import tpu as pltpu
from jax.experimental.pallas import tpu_sc as plsc

B, N, E = 4, 8192, 768
NKEEP = 4915
KPAD = 5120
NH = 7
NHEADS = 4

SR, SC_ = 64, 128

_NC, _NS = 2, 16
NW = _NC * _NS
TOK_W = (B * N) // NW
ROWS_W = (B * KPAD) // NW
CH = 64
NCHUNK = ROWS_W // CH


def _sort_kernel(s_ref, gidx_ref):
    m0 = lax.bitcast_convert_type(s_ref[...], jnp.int32)
    erow = lax.broadcasted_iota(jnp.int32, (B, SR, SC_), 1)
    ecol = lax.broadcasted_iota(jnp.int32, (B, SR, SC_), 2)
    e = erow * SC_ + ecol
    idx0 = e

    def body(_, carry):
        m, idx, k, j = carry
        lo_pos = (e & j) == 0
        up = (e & k) == 0

        def lane_case(ops):
            m_, i_ = ops
            return tuple(
                jnp.where(lo_pos, pltpu.roll(x, SC_ - j, 2),
                          pltpu.roll(x, j, 2))
                for x in (m_, i_))

        def row_case(ops):
            m_, i_ = ops
            jr = lax.shift_right_logical(j, 7)
            return tuple(
                jnp.where(lo_pos, pltpu.roll(x, SR - jr, 1),
                          pltpu.roll(x, jr, 1))
                for x in (m_, i_))

        bm, bidx = lax.cond(j < SC_, lane_case, row_case, (m, idx))
        blta = (bm > m) | ((bm == m) & (bidx < idx))
        want_b = blta ^ (lo_pos != up)
        m = jnp.where(want_b, bm, m)
        idx = jnp.where(want_b, bidx, idx)
        jn = lax.shift_right_logical(j, 1)
        done = jn == 0
        kn = jnp.where(done, lax.shift_left(k, 1), k)
        j2 = jnp.where(done, lax.shift_right_logical(kn, 1), jn)
        return m, idx, kn, j2

    _, idx, _, _ = lax.fori_loop(
        0, 91, body, (m0, idx0, jnp.int32(2), jnp.int32(1)))
    boff = lax.broadcasted_iota(jnp.int32, (B, SR, SC_), 0) * N
    gidx_ref[...] = idx + boff


def _sc_gather_body(xtile, gidx, ctab, xout, cout,
                    idx_v, jx_v, rows_v, ci0_v, ci1_v, o0_v, o1_v,
                    sem0, sem1):
    wid = lax.axis_index("s") * _NC + lax.axis_index("c")
    bi = wid // (NW // B)
    out_base = wid * ROWS_W
    q_base = (wid % (NW // B)) * ROWS_W
    sems = (sem0, sem1)

    for c in range(NCHUNK):
        pltpu.sync_copy(gidx.at[pl.ds(bi * N + q_base + c * CH, CH)],
                        idx_v.at[c])

    for c in range(NCHUNK):
        for k in range(CH // 16):
            iv = idx_v[c, pl.ds(k * 16, 16)]
            ci0_v[c, pl.ds(k * 16, 16)] = iv + bi * N
            ci1_v[c, pl.ds(k * 16, 16)] = iv + (bi + 1) * N
            bse = iv + 40 * (iv >> 3)
            for cc in range(6):
                jx_v[c, cc, pl.ds(k * 16, 16)] = bse + 8 * cc

    def fire(c):
        sem = sems[c % 2]
        ds = [
            pltpu.async_copy(xtile.at[jx_v.at[c, cc]],
                             rows_v.at[c % 2, cc], sem)
            for cc in range(6)
        ]
        ds.append(pltpu.async_copy(ctab.at[ci0_v.at[c]], o0_v.at[c], sem))
        ds.append(pltpu.async_copy(ctab.at[ci1_v.at[c]], o1_v.at[c], sem))
        return ds

    pend = fire(0)
    for c in range(NCHUNK):
        nxt = fire(c + 1) if c + 1 < NCHUNK else None
        for d in pend:
            d.wait()
        for cc in range(6):
            pltpu.sync_copy(
                rows_v.at[c % 2, cc],
                xout.at[pl.ds(out_base + c * CH, CH),
                        pl.ds(cc * 128, 128)])
        pltpu.sync_copy(
            o0_v.at[c],
            cout.at[pl.ds((bi * 2) * KPAD + q_base + c * CH, CH)])
        pltpu.sync_copy(
            o1_v.at[c],
            cout.at[pl.ds((bi * 2 + 1) * KPAD + q_base + c * CH, CH)])
        pend = nxt


@functools.partial(
    pl.kernel,
    mesh=plsc.VectorSubcoreMesh(core_axis_name="c", subcore_axis_name="s"),
    out_type=[
        jax.ShapeDtypeStruct((B * KPAD, E), jnp.float32),
        jax.ShapeDtypeStruct((B * 2 * KPAD,), jnp.float32),
    ],
    scratch_types=[
        pltpu.VMEM((NCHUNK, CH), jnp.int32),
        pltpu.VMEM((NCHUNK, 6, CH), jnp.int32),
        pltpu.VMEM((2, 6, CH, 128), jnp.float32),
        pltpu.VMEM((NCHUNK, CH), jnp.int32),
        pltpu.VMEM((NCHUNK, CH), jnp.int32),
        pltpu.VMEM((NCHUNK, CH), jnp.float32),
        pltpu.VMEM((NCHUNK, CH), jnp.float32),
        pltpu.SemaphoreType.DMA,
        pltpu.SemaphoreType.DMA,
    ],
)
def _sc_gather(xtile, gidx, ctab, xout, cout, *scratch):
    _sc_gather_body(xtile, gidx, ctab, xout, cout, *scratch)


def kernel(x, att_nh, coords, W1, b1, W2, b2, W3, b3, W4, b4, W5, b5):
    b, n, e = x.shape
    bt, n_heads, nh, _ = att_nh.shape

    a = att_nh.reshape(b, n, nh, nh, n_heads)
    h = a @ W1.T + b1
    h = h @ W2.T + b2
    att_vote = h.reshape(b, n, nh * nh)
    v = att_vote @ W3.T + b3
    v = v @ W4.T + b4
    v = v @ W5.T + b5
    v = jnp.squeeze(v)
    s = jax.nn.softmax(v, axis=1)

    gidx = pl.pallas_call(
        _sort_kernel,
        out_shape=jax.ShapeDtypeStruct((B, SR, SC_), jnp.int32),
    )(s.reshape(B, SR, SC_)).reshape(B * N)

    xtile = jnp.transpose(
        x.reshape(B * N // 8, 8, E // 128, 128), (0, 2, 1, 3)
    ).reshape(B * N // 8 * (E // 128) * 8, 128)
    ctab = coords[..., 0].reshape(B * 2 * N)
    xout, cout = _sc_gather(xtile, gidx, ctab)

    x_out = xout.reshape(B, KPAD, E)[:, :NKEEP]
    coords_out = cout.reshape(B, 2, KPAD)[:, :, :NKEEP, None]
    return (x_out, coords_out)


# speedup vs baseline: 2.1720x; 1.0083x over previous
import functools

import jax
import jax.numpy as jnp
from jax import lax
from jax.experimental import pallas as pl
from jax.experimental.pallas import tpu as pltpu
from jax.experimental.pallas import tpu_sc as plsc

B, N, E = 4, 8192, 768
NKEEP = 4915
KPAD = 5120
NH = 7
NHEADS = 4

SR, SC_ = 64, 128

_NC, _NS = 2, 16
NW = _NC * _NS
TOK_W = (B * N) // NW
ROWS_W = (B * KPAD) // NW
CH = 64
NCHUNK = ROWS_W // CH


def _sort_kernel(s_ref, gidx_ref):
    m0 = lax.bitcast_convert_type(s_ref[...], jnp.int32)
    erow = lax.broadcasted_iota(jnp.int32, (B, SR, SC_), 1)
    ecol = lax.broadcasted_iota(jnp.int32, (B, SR, SC_), 2)
    e = erow * SC_ + ecol
    idx0 = e

    def body(_, carry):
        m, idx, k, j = carry
        lo_pos = (e & j) == 0
        up = (e & k) == 0

        def lane_case(ops):
            m_, i_ = ops
            return tuple(
                jnp.where(lo_pos, pltpu.roll(x, SC_ - j, 2),
                          pltpu.roll(x, j, 2))
                for x in (m_, i_))

        def row_case(ops):
            m_, i_ = ops
            jr = lax.shift_right_logical(j, 7)
            return tuple(
                jnp.where(lo_pos, pltpu.roll(x, SR - jr, 1),
                          pltpu.roll(x, jr, 1))
                for x in (m_, i_))

        bm, bidx = lax.cond(j < SC_, lane_case, row_case, (m, idx))
        blta = (bm > m) | ((bm == m) & (bidx < idx))
        want_b = blta ^ (lo_pos != up)
        m = jnp.where(want_b, bm, m)
        idx = jnp.where(want_b, bidx, idx)
        jn = lax.shift_right_logical(j, 1)
        done = jn == 0
        kn = jnp.where(done, lax.shift_left(k, 1), k)
        j2 = jnp.where(done, lax.shift_right_logical(kn, 1), jn)
        return m, idx, kn, j2

    _, idx, _, _ = lax.fori_loop(
        0, 91, body, (m0, idx0, jnp.int32(2), jnp.int32(1)))
    boff = lax.broadcasted_iota(jnp.int32, (B, SR, SC_), 0) * N
    gidx_ref[...] = idx + boff


def _sc_gather_body(xm, gidx, ctab, xout, cout,
                    idx_v, rows_v, ci0_v, ci1_v, o0_v, o1_v,
                    sem0, sem1):
    wid = lax.axis_index("s") * _NC + lax.axis_index("c")
    bi = wid // (NW // B)
    out_base = wid * ROWS_W
    q_base = (wid % (NW // B)) * ROWS_W
    sems = (sem0, sem1)

    pltpu.sync_copy(gidx.at[pl.ds(bi * N + q_base, ROWS_W)], idx_v)

    for k in range(ROWS_W // 16):
        iv = idx_v[pl.ds(k * 16, 16)]
        ci0_v[pl.ds(k * 16, 16)] = iv + bi * N
        ci1_v[pl.ds(k * 16, 16)] = iv + (bi + 1) * N

    def fire(c):
        sem = sems[c % 2]
        return (
            pltpu.async_copy(xm.at[idx_v.at[pl.ds(c * CH, CH)]],
                             rows_v.at[c % 2], sem),
            pltpu.async_copy(ctab.at[ci0_v.at[pl.ds(c * CH, CH)]],
                             o0_v.at[pl.ds(c * CH, CH)], sem),
            pltpu.async_copy(ctab.at[ci1_v.at[pl.ds(c * CH, CH)]],
                             o1_v.at[pl.ds(c * CH, CH)], sem),
        )

    pend = fire(0)
    for c in range(NCHUNK):
        nxt = fire(c + 1) if c + 1 < NCHUNK else None
        for d in pend:
            d.wait()
        pltpu.sync_copy(rows_v.at[c % 2],
                        xout.at[pl.ds(out_base + c * CH, CH)])
        pend = nxt

    pltpu.sync_copy(o0_v, cout.at[pl.ds((bi * 2) * KPAD + q_base, ROWS_W)])
    pltpu.sync_copy(o1_v,
                    cout.at[pl.ds((bi * 2 + 1) * KPAD + q_base, ROWS_W)])


@functools.partial(
    pl.kernel,
    mesh=plsc.VectorSubcoreMesh(core_axis_name="c", subcore_axis_name="s"),
    compiler_params=pltpu.CompilerParams(use_tc_tiling_on_sc=True),
    out_type=[
        jax.ShapeDtypeStruct((B * KPAD, E), jnp.float32),
        jax.ShapeDtypeStruct((B * 2 * KPAD,), jnp.float32),
    ],
    scratch_types=[
        pltpu.VMEM((ROWS_W,), jnp.int32),
        pltpu.VMEM((2, CH, E), jnp.float32),
        pltpu.VMEM((ROWS_W,), jnp.int32),
        pltpu.VMEM((ROWS_W,), jnp.int32),
        pltpu.VMEM((ROWS_W,), jnp.float32),
        pltpu.VMEM((ROWS_W,), jnp.float32),
        pltpu.SemaphoreType.DMA,
        pltpu.SemaphoreType.DMA,
    ],
)
def _sc_gather(xm, gidx, ctab, xout, cout, *scratch):
    _sc_gather_body(xm, gidx, ctab, xout, cout, *scratch)


def kernel(x, att_nh, coords, W1, b1, W2, b2, W3, b3, W4, b4, W5, b5):
    b, n, e = x.shape
    bt, n_heads, nh, _ = att_nh.shape

    a = att_nh.reshape(b, n, nh, nh, n_heads)
    h = a @ W1.T + b1
    h = h @ W2.T + b2
    att_vote = h.reshape(b, n, nh * nh)
    v = att_vote @ W3.T + b3
    v = v @ W4.T + b4
    v = v @ W5.T + b5
    v = jnp.squeeze(v)
    s = jax.nn.softmax(v, axis=1)

    gidx = pl.pallas_call(
        _sort_kernel,
        out_shape=jax.ShapeDtypeStruct((B, SR, SC_), jnp.int32),
    )(s.reshape(B, SR, SC_)).reshape(B * N)

    ctab = coords[..., 0].reshape(B * 2 * N)
    xout, cout = _sc_gather(x.reshape(B * N, E), gidx, ctab)

    x_out = xout.reshape(B, KPAD, E)[:, :NKEEP]
    coords_out = cout.reshape(B, 2, KPAD)[:, :, :NKEEP, None]
    return (x_out, coords_out)
